# baseline (device time: 12867 ns/iter reference)
import jax
import jax.numpy as jnp
from jax import lax
from jax.experimental import pallas as pl
from jax.experimental.pallas import tpu as pltpu

N_DEV = 32
PLANE = 8
NZ = 4


def kernel(x):
    _, n = x.shape

    def body(x_ref, out_ref, buf1, buf2, ssem1, ssem2, rsem1, rsem2, lsem):
        my = lax.axis_index("i")
        myp = lax.rem(my, PLANE)
        myz = my // PLANE
        plane_base = my - myp

        barrier_sem = pltpu.get_barrier_semaphore()
        for q in range(1, PLANE):
            peer = plane_base + lax.rem(myp + q, PLANE)
            pl.semaphore_signal(
                barrier_sem, inc=1,
                device_id=(peer,), device_id_type=pl.DeviceIdType.MESH,
            )
        for r in range(1, NZ):
            peer = myp + PLANE * lax.rem(myz + r, NZ)
            pl.semaphore_signal(
                barrier_sem, inc=1,
                device_id=(peer,), device_id_type=pl.DeviceIdType.MESH,
            )

        out_ref[:, :] = jnp.sum(x_ref[:, :], axis=0, keepdims=True)

        pl.semaphore_wait(barrier_sem, (PLANE - 1) + (NZ - 1))

        loc1 = pltpu.make_async_copy(out_ref, buf1.at[myp], lsem)
        loc1.start()
        sends1 = []
        for q in range(1, PLANE):
            peer = plane_base + lax.rem(myp + q, PLANE)
            rdma = pltpu.make_async_remote_copy(
                src_ref=out_ref,
                dst_ref=buf1.at[myp],
                send_sem=ssem1.at[q - 1],
                recv_sem=rsem1.at[myp],
                device_id=(peer,),
                device_id_type=pl.DeviceIdType.MESH,
            )
            rdma.start()
            sends1.append(rdma)
        loc1.wait()
        for q in range(1, PLANE):
            sender = lax.rem(myp + q, PLANE)
            recv = pltpu.make_async_remote_copy(
                src_ref=out_ref,
                dst_ref=buf1.at[sender],
                send_sem=ssem1.at[q - 1],
                recv_sem=rsem1.at[sender],
                device_id=(my,),
                device_id_type=pl.DeviceIdType.MESH,
            )
            recv.wait_recv()
        for rdma in sends1:
            rdma.wait_send()
        out_ref[:, :] = jnp.sum(buf1[:, :, :], axis=0)

        loc2 = pltpu.make_async_copy(out_ref, buf2.at[myz], lsem)
        loc2.start()
        sends2 = []
        for r in range(1, NZ):
            peer = myp + PLANE * lax.rem(myz + r, NZ)
            rdma = pltpu.make_async_remote_copy(
                src_ref=out_ref,
                dst_ref=buf2.at[myz],
                send_sem=ssem2.at[r - 1],
                recv_sem=rsem2.at[myz],
                device_id=(peer,),
                device_id_type=pl.DeviceIdType.MESH,
            )
            rdma.start()
            sends2.append(rdma)
        loc2.wait()
        for r in range(1, NZ):
            sender = lax.rem(myz + r, NZ)
            recv = pltpu.make_async_remote_copy(
                src_ref=out_ref,
                dst_ref=buf2.at[sender],
                send_sem=ssem2.at[r - 1],
                recv_sem=rsem2.at[sender],
                device_id=(my,),
                device_id_type=pl.DeviceIdType.MESH,
            )
            recv.wait_recv()
        for rdma in sends2:
            rdma.wait_send()
        out_ref[:, :] = jnp.sum(buf2[:, :, :], axis=0)

    return pl.pallas_call(
        body,
        out_shape=jax.ShapeDtypeStruct((1, n), jnp.float32),
        in_specs=[pl.BlockSpec(memory_space=pltpu.VMEM)],
        out_specs=pl.BlockSpec(memory_space=pltpu.VMEM),
        scratch_shapes=[
            pltpu.VMEM((PLANE, 1, n), jnp.float32),
            pltpu.VMEM((NZ, 1, n), jnp.float32),
            pltpu.SemaphoreType.DMA((PLANE - 1,)),
            pltpu.SemaphoreType.DMA((NZ - 1,)),
            pltpu.SemaphoreType.DMA((PLANE,)),
            pltpu.SemaphoreType.DMA((NZ,)),
            pltpu.SemaphoreType.DMA,
        ],
        compiler_params=pltpu.CompilerParams(collective_id=0),
    )(x)


# device time: 12477 ns/iter; 1.0313x vs baseline; 1.0313x over previous
import jax
import jax.numpy as jnp
from jax import lax
from jax.experimental import pallas as pl
from jax.experimental.pallas import tpu as pltpu

N_DEV = 32


def kernel(x):
    _, n = x.shape

    def body(x_ref, out_ref, recv_buf, send_sems, recv_sems, loc_sem):
        my = lax.axis_index("i")

        barrier_sem = pltpu.get_barrier_semaphore()
        for off in range(1, N_DEV):
            peer = lax.rem(my + off, N_DEV)
            pl.semaphore_signal(
                barrier_sem, inc=1,
                device_id=(peer,), device_id_type=pl.DeviceIdType.MESH,
            )

        out_ref[:, :] = jnp.sum(x_ref[:, :], axis=0, keepdims=True)

        loc = pltpu.make_async_copy(out_ref.at[0], recv_buf.at[my], loc_sem)
        loc.start()

        pl.semaphore_wait(barrier_sem, N_DEV - 1)

        sends = []
        for off in range(1, N_DEV):
            peer = lax.rem(my + off, N_DEV)
            rdma = pltpu.make_async_remote_copy(
                src_ref=out_ref.at[0],
                dst_ref=recv_buf.at[my],
                send_sem=send_sems.at[off - 1],
                recv_sem=recv_sems.at[my],
                device_id=(peer,),
                device_id_type=pl.DeviceIdType.MESH,
            )
            rdma.start()
            sends.append(rdma)

        loc.wait()

        for off in range(1, N_DEV):
            sender = lax.rem(my + off, N_DEV)
            recv = pltpu.make_async_remote_copy(
                src_ref=out_ref.at[0],
                dst_ref=recv_buf.at[sender],
                send_sem=send_sems.at[off - 1],
                recv_sem=recv_sems.at[sender],
                device_id=(sender,),
                device_id_type=pl.DeviceIdType.MESH,
            )
            recv.wait_recv()

        out_ref[:, :] = jnp.sum(recv_buf[:, :], axis=0, keepdims=True)

        for rdma in sends:
            rdma.wait_send()

    return pl.pallas_call(
        body,
        out_shape=jax.ShapeDtypeStruct((1, n), jnp.float32),
        in_specs=[pl.BlockSpec(memory_space=pltpu.VMEM)],
        out_specs=pl.BlockSpec(memory_space=pltpu.VMEM),
        scratch_shapes=[
            pltpu.VMEM((N_DEV, n), jnp.float32),
            pltpu.SemaphoreType.DMA((N_DEV - 1,)),
            pltpu.SemaphoreType.DMA((N_DEV,)),
            pltpu.SemaphoreType.DMA,
        ],
        compiler_params=pltpu.CompilerParams(collective_id=0),
    )(x)


# device time: 11916 ns/iter; 1.0798x vs baseline; 1.0471x over previous
import jax
import jax.numpy as jnp
from jax import lax
from jax.experimental import pallas as pl
from jax.experimental.pallas import tpu as pltpu

N_DEV = 32


def kernel(x):
    _, n = x.shape

    def body(x_ref, out_ref, recv_buf, send_sems, recv_sems, loc_sem):
        my = lax.axis_index("i")

        barrier_sem = pltpu.get_barrier_semaphore()
        for off in range(1, N_DEV):
            peer = lax.rem(my + off, N_DEV)
            pl.semaphore_signal(
                barrier_sem, inc=1,
                device_id=(peer,), device_id_type=pl.DeviceIdType.MESH,
            )

        out_ref[:, :] = jnp.sum(x_ref[:, :], axis=0, keepdims=True)

        pl.semaphore_wait(barrier_sem, N_DEV - 1)

        loc = pltpu.make_async_copy(out_ref, recv_buf.at[my], loc_sem)
        loc.start()

        sends = []
        for off in range(1, N_DEV):
            peer = lax.rem(my + off, N_DEV)
            rdma = pltpu.make_async_remote_copy(
                src_ref=out_ref,
                dst_ref=recv_buf.at[my],
                send_sem=send_sems.at[off - 1],
                recv_sem=recv_sems.at[my],
                device_id=(peer,),
                device_id_type=pl.DeviceIdType.MESH,
            )
            rdma.start()
            sends.append(rdma)

        loc.wait()

        for off in range(1, N_DEV):
            sender = lax.rem(my + off, N_DEV)
            recv = pltpu.make_async_remote_copy(
                src_ref=out_ref,
                dst_ref=recv_buf.at[sender],
                send_sem=send_sems.at[off - 1],
                recv_sem=recv_sems.at[sender],
                device_id=(sender,),
                device_id_type=pl.DeviceIdType.MESH,
            )
            recv.wait_recv()

        out_ref[:, :] = jnp.sum(recv_buf[:, :, :], axis=0)

        for rdma in sends:
            rdma.wait_send()

    return pl.pallas_call(
        body,
        out_shape=jax.ShapeDtypeStruct((1, n), jnp.float32),
        in_specs=[pl.BlockSpec(memory_space=pltpu.VMEM)],
        out_specs=pl.BlockSpec(memory_space=pltpu.VMEM),
        scratch_shapes=[
            pltpu.VMEM((N_DEV, 1, n), jnp.float32),
            pltpu.SemaphoreType.DMA((N_DEV - 1,)),
            pltpu.SemaphoreType.DMA((N_DEV,)),
            pltpu.SemaphoreType.DMA,
        ],
        compiler_params=pltpu.CompilerParams(collective_id=0),
    )(x)
